# baseline (speedup 1.0000x reference)
import functools

import jax
import jax.numpy as jnp
from jax import lax
from jax.experimental import pallas as pl
from jax.experimental.pallas import tpu as pltpu
from jax.experimental.pallas import tpu_sc as plsc

N = 10000
E = 320000
NP = 10240
NW = 32
CHUNK = 128
CH = 80
EW = CH * CHUNK
PE = NW * EW
TILES = 16
RPT = NP // TILES
NR = NP // 128
F = 128


def _make_sc_gather_scatter():
    mesh = plsc.VectorSubcoreMesh(core_axis_name="c", subcore_axis_name="s")

    NBUF = 2

    @functools.partial(
        pl.kernel,
        mesh=mesh,
        out_type=jax.ShapeDtypeStruct((2, NP, F), jnp.float32),
        scratch_types=[
            pltpu.VMEM((NBUF, CHUNK), jnp.int32),
            pltpu.VMEM((NBUF, CHUNK), jnp.int32),
            pltpu.VMEM((NBUF, CHUNK, F), jnp.float32),
            pltpu.VMEM_SHARED((NP, F), jnp.float32),
        ] + [pltpu.SemaphoreType.DMA] * (3 * NBUF),
    )
    def k(y_hbm, src_hbm, dst_hbm, out_hbm, srcb, dstb, rows, acc, *sems):
        ssem = sems[0:NBUF]
        dsem = sems[NBUF:2 * NBUF]
        gsem = sems[2 * NBUF:3 * NBUF]
        c = lax.axis_index("c")
        s = lax.axis_index("s")
        wid = s * 2 + c
        pltpu.sync_copy(y_hbm.at[pl.ds(s * RPT, RPT)], acc.at[pl.ds(s * RPT, RPT)])
        plsc.subcore_barrier()

        def body(g, carry):
            sc, dc, gc = [], [], []
            for b in range(NBUF):
                j = g * NBUF + b
                sc.append(pltpu.async_copy(
                    src_hbm.at[wid, pl.ds(j * CHUNK, CHUNK)], srcb.at[b], ssem[b]))
                dc.append(pltpu.async_copy(
                    dst_hbm.at[wid, pl.ds(j * CHUNK, CHUNK)], dstb.at[b], dsem[b]))
            for b in range(NBUF):
                sc[b].wait()
                gc.append(pltpu.async_copy(y_hbm.at[srcb.at[b]], rows.at[b],
                                           gsem[b]))
            for b in range(NBUF):
                gc[b].wait()
                dc[b].wait()
                pltpu.sync_copy(rows.at[b], acc.at[dstb.at[b]], add=True)
            return carry

        lax.fori_loop(0, CH // NBUF, body, 0)
        plsc.subcore_barrier()
        pltpu.sync_copy(acc.at[pl.ds(s * RPT, RPT)],
                        out_hbm.at[c, pl.ds(s * RPT, RPT)])

    return k


def _make_sc_degree():
    mesh = plsc.VectorSubcoreMesh(core_axis_name="c", subcore_axis_name="s")

    @functools.partial(
        pl.kernel,
        mesh=mesh,
        out_type=jax.ShapeDtypeStruct((2, NP, F), jnp.float32),
        scratch_types=[
            pltpu.VMEM((CH, CHUNK), jnp.int32),
            pltpu.VMEM((CHUNK, F), jnp.float32),
            pltpu.VMEM_SHARED((NP, F), jnp.float32),
        ],
    )
    def k(dst_hbm, out_hbm, dstv, rows, acc):
        c = lax.axis_index("c")
        s = lax.axis_index("s")
        wid = s * 2 + c
        pltpu.sync_copy(dst_hbm.at[wid], dstv)

        def zero(t, carry):
            rows[t // 8, pl.ds((t % 8) * 16, 16)] = jnp.zeros((16,), jnp.float32)
            return carry

        lax.fori_loop(0, CHUNK * 8, zero, 0)
        for r in range(RPT // CHUNK):
            pltpu.sync_copy(rows, acc.at[pl.ds(s * RPT + r * CHUNK, CHUNK)])

        def fill(t, carry):
            rows[t // 8, pl.ds((t % 8) * 16, 16)] = jnp.ones((16,), jnp.float32)
            return carry

        lax.fori_loop(0, CHUNK * 8, fill, 0)
        plsc.subcore_barrier()

        def body(j, carry):
            pltpu.sync_copy(rows, acc.at[dstv.at[j]], add=True)
            return carry

        lax.fori_loop(0, CH, body, 0)
        plsc.subcore_barrier()
        pltpu.sync_copy(acc.at[pl.ds(s * RPT, RPT)],
                        out_hbm.at[c, pl.ds(s * RPT, RPT)])

    return k


_sc_layer = _make_sc_gather_scatter()
_sc_degree = _make_sc_degree()


def _tc_prep(xp, W1, d0, d1):
    def body(x_ref, w_ref, d0_ref, d1_ref, y_ref, dinv_ref):
        deg = d0_ref[...] + d1_ref[...] + 1.0
        dinv = lax.rsqrt(deg)
        dinv_ref[...] = dinv
        xw = jnp.dot(x_ref[...], w_ref[...], preferred_element_type=jnp.float32)
        y_ref[...] = dinv * xw

    return pl.pallas_call(
        body,
        out_shape=(jax.ShapeDtypeStruct((NP, F), jnp.float32),
                   jax.ShapeDtypeStruct((NP, 1), jnp.float32)),
    )(xp, W1, d0, d1)


def _tc_layer(sacc, y, dinv, b, W):
    def body(s_ref, y_ref, dinv_ref, b_ref, w_ref, o_ref):
        agg = s_ref[0] + s_ref[1] - y_ref[...]
        h = jax.nn.relu(dinv_ref[...] * agg + b_ref[...])
        o_ref[...] = dinv_ref[...] * jnp.dot(
            h, w_ref[...], preferred_element_type=jnp.float32)

    return pl.pallas_call(
        body,
        out_shape=jax.ShapeDtypeStruct((NP, F), jnp.float32),
    )(sacc, y, dinv, b, W)


def _tc_final(sacc, y, dinv, b, Wm, Wfc, bfc):
    def body(s_ref, y_ref, dinv_ref, b_ref, wm_ref, wfc_ref, bfc_ref, o_ref):
        agg = s_ref[0] + s_ref[1] - y_ref[...]
        h = jax.nn.relu(dinv_ref[...] * agg + b_ref[...])
        rowid = lax.broadcasted_iota(jnp.int32, h.shape, 0)
        h = jnp.where(rowid < N, h, 0.0)
        colsum = jnp.sum(h, axis=0, keepdims=True)
        gc = jnp.dot(colsum / N, wm_ref[...],
                     preferred_element_type=jnp.float32)
        tg = jnp.tanh(gc)
        scores = jax.nn.sigmoid(jnp.sum(h * tg, axis=1, keepdims=True))
        rep = jnp.sum(h * scores, axis=0, keepdims=True)
        o_ref[...] = jax.nn.relu(
            lax.dot_general(rep, wfc_ref[...], (((1,), (1,)), ((), ())),
                            preferred_element_type=jnp.float32)
            + bfc_ref[...])

    return pl.pallas_call(
        body,
        out_shape=jax.ShapeDtypeStruct((1, Wfc.shape[0]), jnp.float32),
    )(sacc, y, dinv, b, Wm, Wfc, bfc)


def _pad2(a, r, c):
    return jnp.pad(a, ((0, r - a.shape[0]), (0, c - a.shape[1])))


def kernel(x, edge_index, W1, b1, W2, b2, W3, b3, Wm, Wfc, bfc):
    ei = edge_index.astype(jnp.int32)
    pad = jnp.full((2, PE - E), N, jnp.int32)
    eip = jnp.concatenate([ei, pad], axis=1)
    src = eip[0].reshape(NW, EW)
    dst = eip[1].reshape(NW, EW)
    dst3 = eip[1].reshape(NW, CH, CHUNK)

    sdeg = _sc_degree(dst3)
    d0 = sdeg[0, :, 0:1]
    d1 = sdeg[1, :, 0:1]

    xp = jnp.pad(x, ((0, NP - N), (0, 0)))
    W2p = _pad2(W2, F, F)
    W3p = _pad2(W3, F, F)
    Wmp = _pad2(Wm, F, F)
    Wfcp = jnp.pad(Wfc, ((0, 0), (0, F - Wfc.shape[1])))
    b1p = jnp.pad(b1, (0, F - b1.shape[0])).reshape(1, F)
    b2p = jnp.pad(b2, (0, F - b2.shape[0])).reshape(1, F)
    b3p = jnp.pad(b3, (0, F - b3.shape[0])).reshape(1, F)

    y1, dinv = _tc_prep(xp, W1, d0, d1)
    s1 = _sc_layer(y1, src, dst)
    y2 = _tc_layer(s1, y1, dinv, b1p, W2p)
    s2 = _sc_layer(y2, src, dst)
    y3 = _tc_layer(s2, y2, dinv, b2p, W3p)
    s3 = _sc_layer(y3, src, dst)
    return _tc_final(s3, y3, dinv, b3p, Wmp, Wfcp, bfc.reshape(1, -1))


# speedup vs baseline: 2.4592x; 2.4592x over previous
import functools

import jax
import jax.numpy as jnp
from jax import lax
from jax.experimental import pallas as pl
from jax.experimental.pallas import tpu as pltpu
from jax.experimental.pallas import tpu_sc as plsc

N = 10000
E = 320000
NP = 10240
NW = 32
CHUNK = 128
CH = 80
EW = CH * CHUNK
PE = NW * EW
TILES = 16
RPT = NP // TILES
NR = NP // 128
F = 128


def _make_sc_gather_scatter():
    mesh = plsc.VectorSubcoreMesh(core_axis_name="c", subcore_axis_name="s")

    NBUF = 2

    @functools.partial(
        pl.kernel,
        mesh=mesh,
        out_type=jax.ShapeDtypeStruct((2, NP, F), jnp.float32),
        scratch_types=[
            pltpu.VMEM((NBUF, CHUNK), jnp.int32),
            pltpu.VMEM((NBUF, CHUNK), jnp.int32),
            pltpu.VMEM((NBUF, CHUNK, F), jnp.float32),
            pltpu.VMEM_SHARED((NP, F), jnp.float32),
        ] + [pltpu.SemaphoreType.DMA] * (3 * NBUF),
    )
    def k(y_hbm, src_hbm, dst_hbm, out_hbm, srcb, dstb, rows, acc, *sems):
        ssem = sems[0:NBUF]
        dsem = sems[NBUF:2 * NBUF]
        gsem = sems[2 * NBUF:3 * NBUF]
        c = lax.axis_index("c")
        s = lax.axis_index("s")
        wid = s * 2 + c
        pltpu.sync_copy(y_hbm.at[pl.ds(s * RPT, RPT)], acc.at[pl.ds(s * RPT, RPT)])
        plsc.subcore_barrier()

        def body(g, carry):
            sc, dc, gc = [], [], []
            for b in range(NBUF):
                j = g * NBUF + b
                sc.append(pltpu.async_copy(
                    src_hbm.at[wid, pl.ds(j * CHUNK, CHUNK)], srcb.at[b], ssem[b]))
                dc.append(pltpu.async_copy(
                    dst_hbm.at[wid, pl.ds(j * CHUNK, CHUNK)], dstb.at[b], dsem[b]))
            for b in range(NBUF):
                sc[b].wait()
                gc.append(pltpu.async_copy(y_hbm.at[srcb.at[b]], rows.at[b],
                                           gsem[b]))
            for b in range(NBUF):
                gc[b].wait()
                dc[b].wait()
                pltpu.sync_copy(rows.at[b], acc.at[dstb.at[b]], add=True)
            return carry

        lax.fori_loop(0, CH // NBUF, body, 0)
        plsc.subcore_barrier()
        pltpu.sync_copy(acc.at[pl.ds(s * RPT, RPT)],
                        out_hbm.at[c, pl.ds(s * RPT, RPT)])

    return k


def _make_sc_degree():
    mesh = plsc.VectorSubcoreMesh(core_axis_name="c", subcore_axis_name="s")

    @functools.partial(
        pl.kernel,
        mesh=mesh,
        out_type=jax.ShapeDtypeStruct((2, NP, F), jnp.float32),
        scratch_types=[
            pltpu.VMEM((CH, CHUNK), jnp.int32),
            pltpu.VMEM((CHUNK, F), jnp.float32),
            pltpu.VMEM_SHARED((NP, F), jnp.float32),
        ],
    )
    def k(dst_hbm, out_hbm, dstv, rows, acc):
        c = lax.axis_index("c")
        s = lax.axis_index("s")
        wid = s * 2 + c
        pltpu.sync_copy(dst_hbm.at[wid], dstv)

        def zero(t, carry):
            rows[t // 8, pl.ds((t % 8) * 16, 16)] = jnp.zeros((16,), jnp.float32)
            return carry

        lax.fori_loop(0, CHUNK * 8, zero, 0)
        for r in range(RPT // CHUNK):
            pltpu.sync_copy(rows, acc.at[pl.ds(s * RPT + r * CHUNK, CHUNK)])

        def fill(t, carry):
            rows[t // 8, pl.ds((t % 8) * 16, 16)] = jnp.ones((16,), jnp.float32)
            return carry

        lax.fori_loop(0, CHUNK * 8, fill, 0)
        plsc.subcore_barrier()

        def body(j, carry):
            pltpu.sync_copy(rows, acc.at[dstv.at[j]], add=True)
            return carry

        lax.fori_loop(0, CH, body, 0)
        plsc.subcore_barrier()
        pltpu.sync_copy(acc.at[pl.ds(s * RPT, RPT)],
                        out_hbm.at[c, pl.ds(s * RPT, RPT)])

    return k


_sc_layer = _make_sc_gather_scatter()
_sc_degree = _make_sc_degree()


def _tc_prep(xp, W1, d0, d1):
    def body(x_ref, w_ref, d0_ref, d1_ref, y_ref, dinv_ref):
        deg = d0_ref[...] + d1_ref[...] + 1.0
        dinv = lax.rsqrt(deg)
        dinv_ref[...] = dinv
        xw = jnp.dot(x_ref[...], w_ref[...], preferred_element_type=jnp.float32)
        y_ref[...] = dinv * xw

    return pl.pallas_call(
        body,
        out_shape=(jax.ShapeDtypeStruct((NP, F), jnp.float32),
                   jax.ShapeDtypeStruct((NP, 1), jnp.float32)),
    )(xp, W1, d0, d1)


def _tc_layer(sacc, y, dinv, b, W):
    def body(s_ref, y_ref, dinv_ref, b_ref, w_ref, o_ref):
        agg = s_ref[0] + s_ref[1] - y_ref[...]
        h = jax.nn.relu(dinv_ref[...] * agg + b_ref[...])
        o_ref[...] = dinv_ref[...] * jnp.dot(
            h, w_ref[...], preferred_element_type=jnp.float32)

    return pl.pallas_call(
        body,
        out_shape=jax.ShapeDtypeStruct((NP, F), jnp.float32),
    )(sacc, y, dinv, b, W)


def _tc_final(sacc, y, dinv, b, Wm, Wfc, bfc):
    def body(s_ref, y_ref, dinv_ref, b_ref, wm_ref, wfc_ref, bfc_ref, o_ref):
        agg = s_ref[0] + s_ref[1] - y_ref[...]
        h = jax.nn.relu(dinv_ref[...] * agg + b_ref[...])
        rowid = lax.broadcasted_iota(jnp.int32, h.shape, 0)
        h = jnp.where(rowid < N, h, 0.0)
        colsum = jnp.sum(h, axis=0, keepdims=True)
        gc = jnp.dot(colsum / N, wm_ref[...],
                     preferred_element_type=jnp.float32)
        tg = jnp.tanh(gc)
        scores = jax.nn.sigmoid(jnp.sum(h * tg, axis=1, keepdims=True))
        rep = jnp.sum(h * scores, axis=0, keepdims=True)
        o_ref[...] = jax.nn.relu(
            lax.dot_general(rep, wfc_ref[...], (((1,), (1,)), ((), ())),
                            preferred_element_type=jnp.float32)
            + bfc_ref[...])

    return pl.pallas_call(
        body,
        out_shape=jax.ShapeDtypeStruct((1, Wfc.shape[0]), jnp.float32),
    )(sacc, y, dinv, b, Wm, Wfc, bfc)


def _pad2(a, r, c):
    return jnp.pad(a, ((0, r - a.shape[0]), (0, c - a.shape[1])))


def kernel(x, edge_index, W1, b1, W2, b2, W3, b3, Wm, Wfc, bfc):
    ei = edge_index.astype(jnp.int32)
    padv = N + jnp.arange(PE - E, dtype=jnp.int32) % (NP - N)
    pad = jnp.stack([padv, padv])
    eip = jnp.concatenate([ei, pad], axis=1)
    src = eip[0].reshape(NW, EW)
    dst = eip[1].reshape(NW, EW)
    dst3 = eip[1].reshape(NW, CH, CHUNK)

    sdeg = _sc_degree(dst3)
    d0 = sdeg[0, :, 0:1]
    d1 = sdeg[1, :, 0:1]

    xp = jnp.pad(x, ((0, NP - N), (0, 0)))
    W2p = _pad2(W2, F, F)
    W3p = _pad2(W3, F, F)
    Wmp = _pad2(Wm, F, F)
    Wfcp = jnp.pad(Wfc, ((0, 0), (0, F - Wfc.shape[1])))
    b1p = jnp.pad(b1, (0, F - b1.shape[0])).reshape(1, F)
    b2p = jnp.pad(b2, (0, F - b2.shape[0])).reshape(1, F)
    b3p = jnp.pad(b3, (0, F - b3.shape[0])).reshape(1, F)

    y1, dinv = _tc_prep(xp, W1, d0, d1)
    s1 = _sc_layer(y1, src, dst)
    y2 = _tc_layer(s1, y1, dinv, b1p, W2p)
    s2 = _sc_layer(y2, src, dst)
    y3 = _tc_layer(s2, y2, dinv, b2p, W3p)
    s3 = _sc_layer(y3, src, dst)
    return _tc_final(s3, y3, dinv, b3p, Wmp, Wfcp, bfc.reshape(1, -1))


# speedup vs baseline: 2.5400x; 1.0328x over previous
import functools

import jax
import jax.numpy as jnp
from jax import lax
from jax.experimental import pallas as pl
from jax.experimental.pallas import tpu as pltpu
from jax.experimental.pallas import tpu_sc as plsc

N = 10000
E = 320000
NP = 10240
NW = 32
CHUNK = 128
CH = 80
EW = CH * CHUNK
PE = NW * EW
TILES = 16
RPT = NP // TILES
NR = NP // 128
F = 128


def _make_sc_gather_scatter():
    mesh = plsc.VectorSubcoreMesh(core_axis_name="c", subcore_axis_name="s")

    NBUF = 2

    @functools.partial(
        pl.kernel,
        mesh=mesh,
        out_type=jax.ShapeDtypeStruct((2, NP, F), jnp.float32),
        scratch_types=[
            pltpu.VMEM((NBUF, CHUNK), jnp.int32),
            pltpu.VMEM((NBUF, CHUNK), jnp.int32),
            pltpu.VMEM((NBUF, CHUNK, F), jnp.float32),
            pltpu.VMEM_SHARED((NP, F), jnp.float32),
        ] + [pltpu.SemaphoreType.DMA] * (4 * NBUF),
    )
    def k(y_hbm, src_hbm, dst_hbm, out_hbm, srcb, dstb, rows, acc, *sems):
        ssem = sems[0:NBUF]
        dsem = sems[NBUF:2 * NBUF]
        gsem = sems[2 * NBUF:3 * NBUF]
        csem = sems[3 * NBUF:4 * NBUF]
        c = lax.axis_index("c")
        s = lax.axis_index("s")
        wid = s * 2 + c
        pltpu.sync_copy(y_hbm.at[pl.ds(s * RPT, RPT)], acc.at[pl.ds(s * RPT, RPT)])
        plsc.subcore_barrier()

        def body(g, carry):
            sc, dc, gc = [], [], []
            for b in range(NBUF):
                @pl.when(g > 0)
                def _drain(b=b):
                    pltpu.make_async_copy(y_hbm.at[pl.ds(0, CHUNK)],
                                          rows.at[b], csem[b]).wait()

                j = g * NBUF + b
                sc.append(pltpu.async_copy(
                    src_hbm.at[wid, pl.ds(j * CHUNK, CHUNK)], srcb.at[b], ssem[b]))
                dc.append(pltpu.async_copy(
                    dst_hbm.at[wid, pl.ds(j * CHUNK, CHUNK)], dstb.at[b], dsem[b]))
            for b in range(NBUF):
                sc[b].wait()
                gc.append(pltpu.async_copy(y_hbm.at[srcb.at[b]], rows.at[b],
                                           gsem[b]))
            for b in range(NBUF):
                gc[b].wait()
                dc[b].wait()
                pltpu.async_copy(rows.at[b], acc.at[dstb.at[b]], csem[b],
                                 add=True)
            return carry

        lax.fori_loop(0, CH // NBUF, body, 0)
        for b in range(NBUF):
            pltpu.make_async_copy(y_hbm.at[pl.ds(0, CHUNK)],
                                  rows.at[b], csem[b]).wait()
        plsc.subcore_barrier()
        pltpu.sync_copy(acc.at[pl.ds(s * RPT, RPT)],
                        out_hbm.at[c, pl.ds(s * RPT, RPT)])

    return k


def _make_sc_degree():
    mesh = plsc.VectorSubcoreMesh(core_axis_name="c", subcore_axis_name="s")

    @functools.partial(
        pl.kernel,
        mesh=mesh,
        out_type=jax.ShapeDtypeStruct((2, NP, F), jnp.float32),
        scratch_types=[
            pltpu.VMEM((CH, CHUNK), jnp.int32),
            pltpu.VMEM((CHUNK, F), jnp.float32),
            pltpu.VMEM_SHARED((NP, F), jnp.float32),
        ],
    )
    def k(dst_hbm, out_hbm, dstv, rows, acc):
        c = lax.axis_index("c")
        s = lax.axis_index("s")
        wid = s * 2 + c
        pltpu.sync_copy(dst_hbm.at[wid], dstv)

        def zero(t, carry):
            rows[t // 8, pl.ds((t % 8) * 16, 16)] = jnp.zeros((16,), jnp.float32)
            return carry

        lax.fori_loop(0, CHUNK * 8, zero, 0)
        for r in range(RPT // CHUNK):
            pltpu.sync_copy(rows, acc.at[pl.ds(s * RPT + r * CHUNK, CHUNK)])

        def fill(t, carry):
            rows[t // 8, pl.ds((t % 8) * 16, 16)] = jnp.ones((16,), jnp.float32)
            return carry

        lax.fori_loop(0, CHUNK * 8, fill, 0)
        plsc.subcore_barrier()

        def body(j, carry):
            pltpu.sync_copy(rows, acc.at[dstv.at[j]], add=True)
            return carry

        lax.fori_loop(0, CH, body, 0)
        plsc.subcore_barrier()
        pltpu.sync_copy(acc.at[pl.ds(s * RPT, RPT)],
                        out_hbm.at[c, pl.ds(s * RPT, RPT)])

    return k


_sc_layer = _make_sc_gather_scatter()
_sc_degree = _make_sc_degree()


def _tc_prep(xp, W1, d0, d1):
    def body(x_ref, w_ref, d0_ref, d1_ref, y_ref, dinv_ref):
        deg = d0_ref[...] + d1_ref[...] + 1.0
        dinv = lax.rsqrt(deg)
        dinv_ref[...] = dinv
        xw = jnp.dot(x_ref[...], w_ref[...], preferred_element_type=jnp.float32)
        y_ref[...] = dinv * xw

    return pl.pallas_call(
        body,
        out_shape=(jax.ShapeDtypeStruct((NP, F), jnp.float32),
                   jax.ShapeDtypeStruct((NP, 1), jnp.float32)),
    )(xp, W1, d0, d1)


def _tc_layer(sacc, y, dinv, b, W):
    def body(s_ref, y_ref, dinv_ref, b_ref, w_ref, o_ref):
        agg = s_ref[0] + s_ref[1] - y_ref[...]
        h = jax.nn.relu(dinv_ref[...] * agg + b_ref[...])
        o_ref[...] = dinv_ref[...] * jnp.dot(
            h, w_ref[...], preferred_element_type=jnp.float32)

    return pl.pallas_call(
        body,
        out_shape=jax.ShapeDtypeStruct((NP, F), jnp.float32),
    )(sacc, y, dinv, b, W)


def _tc_final(sacc, y, dinv, b, Wm, Wfc, bfc):
    def body(s_ref, y_ref, dinv_ref, b_ref, wm_ref, wfc_ref, bfc_ref, o_ref):
        agg = s_ref[0] + s_ref[1] - y_ref[...]
        h = jax.nn.relu(dinv_ref[...] * agg + b_ref[...])
        rowid = lax.broadcasted_iota(jnp.int32, h.shape, 0)
        h = jnp.where(rowid < N, h, 0.0)
        colsum = jnp.sum(h, axis=0, keepdims=True)
        gc = jnp.dot(colsum / N, wm_ref[...],
                     preferred_element_type=jnp.float32)
        tg = jnp.tanh(gc)
        scores = jax.nn.sigmoid(jnp.sum(h * tg, axis=1, keepdims=True))
        rep = jnp.sum(h * scores, axis=0, keepdims=True)
        o_ref[...] = jax.nn.relu(
            lax.dot_general(rep, wfc_ref[...], (((1,), (1,)), ((), ())),
                            preferred_element_type=jnp.float32)
            + bfc_ref[...])

    return pl.pallas_call(
        body,
        out_shape=jax.ShapeDtypeStruct((1, Wfc.shape[0]), jnp.float32),
    )(sacc, y, dinv, b, Wm, Wfc, bfc)


def _pad2(a, r, c):
    return jnp.pad(a, ((0, r - a.shape[0]), (0, c - a.shape[1])))


def kernel(x, edge_index, W1, b1, W2, b2, W3, b3, Wm, Wfc, bfc):
    ei = edge_index.astype(jnp.int32)
    padv = N + jnp.arange(PE - E, dtype=jnp.int32) % (NP - N)
    pad = jnp.stack([padv, padv])
    eip = jnp.concatenate([ei, pad], axis=1)
    src = eip[0].reshape(NW, EW)
    dst = eip[1].reshape(NW, EW)
    dst3 = eip[1].reshape(NW, CH, CHUNK)

    sdeg = _sc_degree(dst3)
    d0 = sdeg[0, :, 0:1]
    d1 = sdeg[1, :, 0:1]

    xp = jnp.pad(x, ((0, NP - N), (0, 0)))
    W2p = _pad2(W2, F, F)
    W3p = _pad2(W3, F, F)
    Wmp = _pad2(Wm, F, F)
    Wfcp = jnp.pad(Wfc, ((0, 0), (0, F - Wfc.shape[1])))
    b1p = jnp.pad(b1, (0, F - b1.shape[0])).reshape(1, F)
    b2p = jnp.pad(b2, (0, F - b2.shape[0])).reshape(1, F)
    b3p = jnp.pad(b3, (0, F - b3.shape[0])).reshape(1, F)

    y1, dinv = _tc_prep(xp, W1, d0, d1)
    s1 = _sc_layer(y1, src, dst)
    y2 = _tc_layer(s1, y1, dinv, b1p, W2p)
    s2 = _sc_layer(y2, src, dst)
    y3 = _tc_layer(s2, y2, dinv, b2p, W3p)
    s3 = _sc_layer(y3, src, dst)
    return _tc_final(s3, y3, dinv, b3p, Wmp, Wfcp, bfc.reshape(1, -1))


# speedup vs baseline: 2.8071x; 1.1052x over previous
import functools

import jax
import jax.numpy as jnp
from jax import lax
from jax.experimental import pallas as pl
from jax.experimental.pallas import tpu as pltpu
from jax.experimental.pallas import tpu_sc as plsc

N = 10000
E = 320000
NP = 10240
NW = 32
CHUNK = 128
CH = 80
EW = CH * CHUNK
PE = NW * EW
TILES = 16
RPT = NP // TILES
NR = NP // 128
F = 128


def _make_sc_gather_scatter():
    mesh = plsc.VectorSubcoreMesh(core_axis_name="c", subcore_axis_name="s")

    NBUF = 2

    @functools.partial(
        pl.kernel,
        mesh=mesh,
        out_type=jax.ShapeDtypeStruct((2, NP, F), jnp.float32),
        scratch_types=[
            pltpu.VMEM((NBUF, CHUNK), jnp.int32),
            pltpu.VMEM((NBUF, CHUNK), jnp.int32),
            pltpu.VMEM((NBUF, CHUNK, F), jnp.float32),
            pltpu.VMEM_SHARED((NP, F), jnp.float32),
        ] + [pltpu.SemaphoreType.DMA] * (4 * NBUF),
    )
    def k(y_hbm, src_hbm, dst_hbm, out_hbm, srcb, dstb, rows, acc, *sems):
        ssem = sems[0:NBUF]
        dsem = sems[NBUF:2 * NBUF]
        gsem = sems[2 * NBUF:3 * NBUF]
        csem = sems[3 * NBUF:4 * NBUF]
        c = lax.axis_index("c")
        s = lax.axis_index("s")
        wid = s * 2 + c
        pltpu.sync_copy(y_hbm.at[pl.ds(s * RPT, RPT)], acc.at[pl.ds(s * RPT, RPT)])
        plsc.subcore_barrier()

        def body(g, carry):
            sc, dc, gc = [], [], []
            for b in range(NBUF):
                @pl.when(g > 0)
                def _drain(b=b):
                    pltpu.make_async_copy(y_hbm.at[pl.ds(0, CHUNK)],
                                          rows.at[b], csem[b]).wait()

                j = g * NBUF + b
                sc.append(pltpu.async_copy(
                    src_hbm.at[wid, pl.ds(j * CHUNK, CHUNK)], srcb.at[b], ssem[b]))
                dc.append(pltpu.async_copy(
                    dst_hbm.at[wid, pl.ds(j * CHUNK, CHUNK)], dstb.at[b], dsem[b]))
            for b in range(NBUF):
                sc[b].wait()
                gc.append(pltpu.async_copy(y_hbm.at[srcb.at[b]], rows.at[b],
                                           gsem[b]))
            for b in range(NBUF):
                gc[b].wait()
                dc[b].wait()
                pltpu.async_copy(rows.at[b], acc.at[dstb.at[b]], csem[b],
                                 add=True)
            return carry

        lax.fori_loop(0, CH // NBUF, body, 0)
        for b in range(NBUF):
            pltpu.make_async_copy(y_hbm.at[pl.ds(0, CHUNK)],
                                  rows.at[b], csem[b]).wait()
        plsc.subcore_barrier()
        pltpu.sync_copy(acc.at[pl.ds(s * RPT, RPT)],
                        out_hbm.at[c, pl.ds(s * RPT, RPT)])

    return k


def _make_sc_degree():
    mesh = plsc.VectorSubcoreMesh(core_axis_name="c", subcore_axis_name="s")

    @functools.partial(
        pl.kernel,
        mesh=mesh,
        out_type=jax.ShapeDtypeStruct((2, NP), jnp.float32),
        scratch_types=[
            pltpu.VMEM((CH, CHUNK), jnp.int32),
            pltpu.VMEM((RPT,), jnp.float32),
            pltpu.VMEM_SHARED((NP,), jnp.float32),
        ],
    )
    def k(dst_hbm, out_hbm, dstv, ones, acc):
        c = lax.axis_index("c")
        s = lax.axis_index("s")
        wid = s * 2 + c
        pltpu.sync_copy(dst_hbm.at[wid], dstv)

        def zero(t, carry):
            ones[pl.ds(t * 16, 16)] = jnp.zeros((16,), jnp.float32)
            return carry

        lax.fori_loop(0, RPT // 16, zero, 0)
        pltpu.sync_copy(ones, acc.at[pl.ds(s * RPT, RPT)])

        def fill(t, carry):
            ones[pl.ds(t * 16, 16)] = jnp.ones((16,), jnp.float32)
            return carry

        lax.fori_loop(0, RPT // 16, fill, 0)
        plsc.subcore_barrier()

        def body(j, carry):
            pltpu.sync_copy(ones.at[pl.ds(0, CHUNK)], acc.at[dstv.at[j]],
                            add=True)
            return carry

        lax.fori_loop(0, CH, body, 0)
        plsc.subcore_barrier()
        pltpu.sync_copy(acc.at[pl.ds(s * RPT, RPT)],
                        out_hbm.at[c, pl.ds(s * RPT, RPT)])

    return k


_sc_layer = _make_sc_gather_scatter()
_sc_degree = _make_sc_degree()


def _tc_prep(xp, W1, d0, d1):
    def body(x_ref, w_ref, d0_ref, d1_ref, y_ref, dinv_ref):
        deg = d0_ref[...] + d1_ref[...] + 1.0
        dinv = lax.rsqrt(deg)
        dinv_ref[...] = dinv
        xw = jnp.dot(x_ref[...], w_ref[...], preferred_element_type=jnp.float32)
        y_ref[...] = dinv * xw

    return pl.pallas_call(
        body,
        out_shape=(jax.ShapeDtypeStruct((NP, F), jnp.float32),
                   jax.ShapeDtypeStruct((NP, 1), jnp.float32)),
    )(xp, W1, d0, d1)


def _tc_layer(sacc, y, dinv, b, W):
    def body(s_ref, y_ref, dinv_ref, b_ref, w_ref, o_ref):
        agg = s_ref[0] + s_ref[1] - y_ref[...]
        h = jax.nn.relu(dinv_ref[...] * agg + b_ref[...])
        o_ref[...] = dinv_ref[...] * jnp.dot(
            h, w_ref[...], preferred_element_type=jnp.float32)

    return pl.pallas_call(
        body,
        out_shape=jax.ShapeDtypeStruct((NP, F), jnp.float32),
    )(sacc, y, dinv, b, W)


def _tc_final(sacc, y, dinv, b, Wm, Wfc, bfc):
    def body(s_ref, y_ref, dinv_ref, b_ref, wm_ref, wfc_ref, bfc_ref, o_ref):
        agg = s_ref[0] + s_ref[1] - y_ref[...]
        h = jax.nn.relu(dinv_ref[...] * agg + b_ref[...])
        rowid = lax.broadcasted_iota(jnp.int32, h.shape, 0)
        h = jnp.where(rowid < N, h, 0.0)
        colsum = jnp.sum(h, axis=0, keepdims=True)
        gc = jnp.dot(colsum / N, wm_ref[...],
                     preferred_element_type=jnp.float32)
        tg = jnp.tanh(gc)
        scores = jax.nn.sigmoid(jnp.sum(h * tg, axis=1, keepdims=True))
        rep = jnp.sum(h * scores, axis=0, keepdims=True)
        o_ref[...] = jax.nn.relu(
            lax.dot_general(rep, wfc_ref[...], (((1,), (1,)), ((), ())),
                            preferred_element_type=jnp.float32)
            + bfc_ref[...])

    return pl.pallas_call(
        body,
        out_shape=jax.ShapeDtypeStruct((1, Wfc.shape[0]), jnp.float32),
    )(sacc, y, dinv, b, Wm, Wfc, bfc)


def _pad2(a, r, c):
    return jnp.pad(a, ((0, r - a.shape[0]), (0, c - a.shape[1])))


def kernel(x, edge_index, W1, b1, W2, b2, W3, b3, Wm, Wfc, bfc):
    ei = edge_index.astype(jnp.int32)
    padv = N + jnp.arange(PE - E, dtype=jnp.int32) % (NP - N)
    pad = jnp.stack([padv, padv])
    eip = jnp.concatenate([ei, pad], axis=1)
    src = eip[0].reshape(NW, EW)
    dst = eip[1].reshape(NW, EW)
    dst3 = eip[1].reshape(NW, CH, CHUNK)

    sdeg = _sc_degree(dst3)
    d0 = sdeg[0].reshape(NP, 1)
    d1 = sdeg[1].reshape(NP, 1)

    xp = jnp.pad(x, ((0, NP - N), (0, 0)))
    W2p = _pad2(W2, F, F)
    W3p = _pad2(W3, F, F)
    Wmp = _pad2(Wm, F, F)
    Wfcp = jnp.pad(Wfc, ((0, 0), (0, F - Wfc.shape[1])))
    b1p = jnp.pad(b1, (0, F - b1.shape[0])).reshape(1, F)
    b2p = jnp.pad(b2, (0, F - b2.shape[0])).reshape(1, F)
    b3p = jnp.pad(b3, (0, F - b3.shape[0])).reshape(1, F)

    y1, dinv = _tc_prep(xp, W1, d0, d1)
    s1 = _sc_layer(y1, src, dst)
    y2 = _tc_layer(s1, y1, dinv, b1p, W2p)
    s2 = _sc_layer(y2, src, dst)
    y3 = _tc_layer(s2, y2, dinv, b2p, W3p)
    s3 = _sc_layer(y3, src, dst)
    return _tc_final(s3, y3, dinv, b3p, Wmp, Wfcp, bfc.reshape(1, -1))
